# Initial kernel scaffold; baseline (speedup 1.0000x reference)
import jax
import jax.numpy as jnp
from jax.experimental import pallas as pl


def kernel(wide_fea1_idx, wide_fea2_idx, deep_fea1_idx, deep_fea2_idx, emb_table, W_wide, b_wide, W1, b1, W2, b2, W3, b3):
    raise NotImplementedError("write your pallas kernel here")


# speedup vs baseline: 10.6166x; 10.6166x over previous
import functools

import jax
import jax.numpy as jnp
from jax import lax
from jax.experimental import pallas as pl
from jax.experimental.pallas import tpu as pltpu
from jax.experimental.pallas import tpu_sc as plsc

_B = 4096
_BINS = 10000
_D = 64
_LW = 10
_LD = 20
_NC = 2
_NS = 16
_NW = _NC * _NS
_RPT = _B // _NW


def _sc_body(w1T, w2T, d1T, d2T, table, wv_hbm, deep_o, wide_o,
             wv, w1i, w2i, d1i, d2i, bufA, bufB, acc, ws, semA, semB):
    wid = lax.axis_index("s") * _NC + lax.axis_index("c")
    base = wid * _RPT

    pltpu.sync_copy(wv_hbm, wv)
    pltpu.sync_copy(w1T.at[:, pl.ds(base, _RPT)], w1i)
    pltpu.sync_copy(w2T.at[:, pl.ds(base, _RPT)], w2i)
    pltpu.sync_copy(d1T.at[:, pl.ds(base, _RPT)], d1i)
    pltpu.sync_copy(d2T.at[:, pl.ds(base, _RPT)], d2i)

    seq = [(d1i, j, 0) for j in range(_LD)] + [(d2i, j, _D) for j in range(_LD)]
    bufs = (bufA, bufB)
    sems = (semA, semB)
    pending = [None, None]
    pending[0] = pltpu.async_copy(table.at[seq[0][0].at[seq[0][1]]], bufs[0], sems[0])
    for k in range(len(seq)):
        if k + 1 < len(seq):
            nref, nj, _ = seq[k + 1]
            pending[(k + 1) % 2] = pltpu.async_copy(
                table.at[nref.at[nj]], bufs[(k + 1) % 2], sems[(k + 1) % 2])
        pending[k % 2].wait()
        buf = bufs[k % 2]
        _, j, coff = seq[k]
        first = j == 0

        def _acc_row(r, carry, buf=buf, coff=coff, first=first):
            for c in range(_D // 16):
                v = buf[r, pl.ds(16 * c, 16)]
                if first:
                    acc[r, pl.ds(coff + 16 * c, 16)] = v
                else:
                    acc[r, pl.ds(coff + 16 * c, 16)] = acc[r, pl.ds(coff + 16 * c, 16)] + v
            return carry

        lax.fori_loop(0, _RPT, _acc_row, 0)

    def _wide_grp(g, carry):
        tot = None
        for iref, off in ((w1i, 0), (w2i, _BINS)):
            ivs = []
            tot_f = None
            for j in range(_LW):
                iv = iref[j, pl.ds(g * 16, 16)]
                if off:
                    iv = iv + off
                v = plsc.load_gather(wv, [iv])
                if j == 0:
                    tot_f = v
                else:
                    keep = iv != ivs[0]
                    for p in ivs[1:]:
                        keep = jnp.logical_and(keep, iv != p)
                    tot_f = tot_f + jnp.where(keep, v, 0.0)
                ivs.append(iv)
            tot = tot_f if tot is None else tot + tot_f
        ws[pl.ds(g * 16, 16)] = tot
        return carry

    lax.fori_loop(0, _RPT // 16, _wide_grp, 0)

    pltpu.sync_copy(acc, deep_o.at[pl.ds(base, _RPT), :])
    pltpu.sync_copy(ws, wide_o.at[pl.ds(base, _RPT)])


@functools.partial(
    pl.kernel,
    out_type=(jax.ShapeDtypeStruct((_B, 2 * _D), jnp.float32),
              jax.ShapeDtypeStruct((_B,), jnp.float32)),
    mesh=plsc.VectorSubcoreMesh(core_axis_name="c", subcore_axis_name="s"),
    compiler_params=pltpu.CompilerParams(
        needs_layout_passes=False, use_tc_tiling_on_sc=False),
    scratch_types=(
        pltpu.VMEM((2 * _BINS,), jnp.float32),
        pltpu.VMEM((_LW, _RPT), jnp.int32),
        pltpu.VMEM((_LW, _RPT), jnp.int32),
        pltpu.VMEM((_LD, _RPT), jnp.int32),
        pltpu.VMEM((_LD, _RPT), jnp.int32),
        pltpu.VMEM((_RPT, _D), jnp.float32),
        pltpu.VMEM((_RPT, _D), jnp.float32),
        pltpu.VMEM((_RPT, 2 * _D), jnp.float32),
        pltpu.VMEM((_RPT,), jnp.float32),
        pltpu.SemaphoreType.DMA,
        pltpu.SemaphoreType.DMA,
    ),
)
def _sc_sparse(*args):
    _sc_body(*args)


def _mlp_body(x_ref, wide_ref, W1_ref, b1_ref, W2_ref, b2_ref, W3_ref, bc_ref, o_ref):
    x = x_ref[...]
    W1s = W1_ref[...] * (1.0 / _LD)
    h = jnp.maximum(jnp.dot(x, W1s, preferred_element_type=jnp.float32) + b1_ref[...], 0.0)
    h = jnp.maximum(jnp.dot(h, W2_ref[...], preferred_element_type=jnp.float32) + b2_ref[...], 0.0)
    deep = jnp.dot(h, W3_ref[...], preferred_element_type=jnp.float32)
    z = 0.5 * (deep + wide_ref[...] + bc_ref[...])
    o_ref[...] = 1.0 / (1.0 + jnp.exp(-z))


def kernel(wide_fea1_idx, wide_fea2_idx, deep_fea1_idx, deep_fea2_idx,
           emb_table, W_wide, b_wide, W1, b1, W2, b2, W3, b3):
    w1T = wide_fea1_idx.astype(jnp.int32).T
    w2T = wide_fea2_idx.astype(jnp.int32).T
    d1T = deep_fea1_idx.astype(jnp.int32).T
    d2T = deep_fea2_idx.astype(jnp.int32).T
    wv = W_wide[:, 0]

    deep_sum, wide_sum = _sc_sparse(w1T, w2T, d1T, d2T, emb_table, wv)

    bc = (b3 + b_wide).reshape(1, 1)
    out = pl.pallas_call(
        _mlp_body,
        out_shape=jax.ShapeDtypeStruct((_B, 1), jnp.float32),
    )(deep_sum, wide_sum.reshape(_B, 1), W1, b1.reshape(1, -1),
      W2, b2.reshape(1, -1), W3, bc)
    return out


# speedup vs baseline: 16.4563x; 1.5501x over previous
import functools

import jax
import jax.numpy as jnp
from jax import lax
from jax.experimental import pallas as pl
from jax.experimental.pallas import tpu as pltpu
from jax.experimental.pallas import tpu_sc as plsc

_B = 4096
_BINS = 10000
_D = 64
_LW = 10
_LD = 20
_NC = 2
_NS = 16
_NW = _NC * _NS
_RPT = _B // _NW


_CH = 5


def _sc_body(w1T, w2T, d1T, d2T, table, wv_hbm, deep_o, wide_o,
             wv, w1i, w2i, d1i, d2i, acc, ws, semA, semB, *bufs):
    wid = lax.axis_index("s") * _NC + lax.axis_index("c")
    base = wid * _RPT

    pltpu.sync_copy(d1T.at[:, pl.ds(base, _RPT)], d1i)
    pltpu.sync_copy(d2T.at[:, pl.ds(base, _RPT)], d2i)

    chunks = ([(d1i, 0, j0) for j0 in range(0, _LD, _CH)]
              + [(d2i, _D, j0) for j0 in range(0, _LD, _CH)])
    sems = (semA, semB)

    def _fire(k):
        iref, _, j0 = chunks[k]
        bank = (k % 2) * _CH
        return [pltpu.async_copy(table.at[iref.at[j0 + jj]], bufs[bank + jj],
                                 sems[k % 2])
                for jj in range(_CH)]

    pending = [None, None]
    pending[0] = _fire(0)
    pending[1] = _fire(1)

    pltpu.sync_copy(w1T.at[:, pl.ds(base, _RPT)], w1i)
    pltpu.sync_copy(w2T.at[:, pl.ds(base, _RPT)], w2i)
    pltpu.sync_copy(wv_hbm, wv)

    for k in range(len(chunks)):
        for d in pending[k % 2]:
            d.wait()
        _, coff, j0 = chunks[k]
        bank = (k % 2) * _CH
        first = j0 == 0

        def _acc_row(r, carry, bank=bank, coff=coff, first=first):
            for c in range(_D // 16):
                s = bufs[bank][r, pl.ds(16 * c, 16)]
                for jj in range(1, _CH):
                    s = s + bufs[bank + jj][r, pl.ds(16 * c, 16)]
                if first:
                    acc[r, pl.ds(coff + 16 * c, 16)] = s
                else:
                    acc[r, pl.ds(coff + 16 * c, 16)] = acc[r, pl.ds(coff + 16 * c, 16)] + s
            return carry

        lax.fori_loop(0, _RPT, _acc_row, 0)
        if k + 2 < len(chunks):
            pending[k % 2] = _fire(k + 2)

    def _wide_grp(g, carry):
        tot = None
        for iref, off in ((w1i, 0), (w2i, _BINS)):
            ivs = []
            tot_f = None
            for j in range(_LW):
                iv = iref[j, pl.ds(g * 16, 16)]
                if off:
                    iv = iv + off
                v = plsc.load_gather(wv, [iv])
                if j == 0:
                    tot_f = v
                else:
                    keep = iv != ivs[0]
                    for p in ivs[1:]:
                        keep = jnp.logical_and(keep, iv != p)
                    tot_f = tot_f + jnp.where(keep, v, 0.0)
                ivs.append(iv)
            tot = tot_f if tot is None else tot + tot_f
        ws[pl.ds(g * 16, 16)] = tot
        return carry

    lax.fori_loop(0, _RPT // 16, _wide_grp, 0)

    pltpu.sync_copy(acc, deep_o.at[pl.ds(base, _RPT), :])
    pltpu.sync_copy(ws, wide_o.at[pl.ds(base, _RPT)])


@functools.partial(
    pl.kernel,
    out_type=(jax.ShapeDtypeStruct((_B, 2 * _D), jnp.float32),
              jax.ShapeDtypeStruct((_B,), jnp.float32)),
    mesh=plsc.VectorSubcoreMesh(core_axis_name="c", subcore_axis_name="s"),
    compiler_params=pltpu.CompilerParams(
        needs_layout_passes=False, use_tc_tiling_on_sc=False),
    scratch_types=(
        pltpu.VMEM((2 * _BINS,), jnp.float32),
        pltpu.VMEM((_LW, _RPT), jnp.int32),
        pltpu.VMEM((_LW, _RPT), jnp.int32),
        pltpu.VMEM((_LD, _RPT), jnp.int32),
        pltpu.VMEM((_LD, _RPT), jnp.int32),
        pltpu.VMEM((_RPT, 2 * _D), jnp.float32),
        pltpu.VMEM((_RPT,), jnp.float32),
        pltpu.SemaphoreType.DMA,
        pltpu.SemaphoreType.DMA,
    ) + tuple(pltpu.VMEM((_RPT, _D), jnp.float32) for _ in range(2 * _CH)),
)
def _sc_sparse(*args):
    _sc_body(*args)


def _mlp_body(x_ref, wide_ref, W1_ref, b1_ref, W2_ref, b2_ref, W3_ref, bc_ref, o_ref):
    x = x_ref[...]
    W1s = W1_ref[...] * (1.0 / _LD)
    h = jnp.maximum(jnp.dot(x, W1s, preferred_element_type=jnp.float32) + b1_ref[...], 0.0)
    h = jnp.maximum(jnp.dot(h, W2_ref[...], preferred_element_type=jnp.float32) + b2_ref[...], 0.0)
    deep = jnp.dot(h, W3_ref[...], preferred_element_type=jnp.float32)
    z = 0.5 * (deep + wide_ref[...] + bc_ref[...])
    o_ref[...] = 1.0 / (1.0 + jnp.exp(-z))


def kernel(wide_fea1_idx, wide_fea2_idx, deep_fea1_idx, deep_fea2_idx,
           emb_table, W_wide, b_wide, W1, b1, W2, b2, W3, b3):
    w1T = wide_fea1_idx.astype(jnp.int32).T
    w2T = wide_fea2_idx.astype(jnp.int32).T
    d1T = deep_fea1_idx.astype(jnp.int32).T
    d2T = deep_fea2_idx.astype(jnp.int32).T
    wv = W_wide[:, 0]

    deep_sum, wide_sum = _sc_sparse(w1T, w2T, d1T, d2T, emb_table, wv)

    bc = (b3 + b_wide).reshape(1, 1)
    out = pl.pallas_call(
        _mlp_body,
        out_shape=jax.ShapeDtypeStruct((_B, 1), jnp.float32),
    )(deep_sum, wide_sum.reshape(_B, 1), W1, b1.reshape(1, -1),
      W2, b2.reshape(1, -1), W3, bc)
    return out
